# baseline (speedup 1.0000x reference)
import jax
import jax.numpy as jnp
from jax import lax
from jax.experimental import pallas as pl
from jax.experimental.pallas import tpu as pltpu
from jax.experimental.pallas import tpu_sc as plsc

N = 10000
E = 320000
D = 128
C = 40

NC = 2
NS = 16
NW = NC * NS

NPAD = 10240
K = 128
CPW = 80
CPW_H = 40
EP = NW * K * CPW

DEG_CHUNK = 2000
EPW_DEG = E // NW

BN = 1024
GRID = (NPAD // BN,)

_mesh = plsc.VectorSubcoreMesh(
    core_axis_name="c", subcore_axis_name="s", num_cores=NC, num_subcores=NS
)
_sc_params = pltpu.CompilerParams(needs_layout_passes=False)


def _deg_body(dst_hbm, out_hbm, dacc, dchunk):
    c = lax.axis_index("c")
    s = lax.axis_index("s")
    wid = c * NS + s
    zeros16 = jnp.zeros((16,), jnp.float32)
    ones16 = jnp.ones((16,), jnp.float32)

    def zb(i, carry):
        dacc[pl.ds(i * 16, 16)] = zeros16
        return carry

    lax.fori_loop(0, NPAD // 16, zb, 0)

    def cb(ci, carry):
        base = wid * EPW_DEG + ci * DEG_CHUNK
        pltpu.sync_copy(dst_hbm.at[pl.ds(base, DEG_CHUNK)], dchunk)

        def ib(j, carry2):
            idx = dchunk[pl.ds(j * 16, 16)]
            plsc.addupdate_scatter(dacc, [idx], ones16)
            return carry2

        lax.fori_loop(0, DEG_CHUNK // 16, ib, 0)
        return carry

    lax.fori_loop(0, EPW_DEG // DEG_CHUNK, cb, 0)
    pltpu.sync_copy(dacc, out_hbm.at[pl.ds(wid * NPAD, NPAD)])


_deg_kernel = pl.kernel(
    _deg_body,
    out_type=jax.ShapeDtypeStruct((NW * NPAD,), jnp.float32),
    mesh=_mesh,
    scratch_types=[
        pltpu.VMEM((NPAD,), jnp.float32),
        pltpu.VMEM((DEG_CHUNK,), jnp.int32),
    ],
    compiler_params=_sc_params,
)


def _agg_body(ga_hbm, gb_hbm, srcp_hbm, dstp_hbm, out_hbm, acc, sidx, didx,
              rows_a, rows_b, sem_a, sem_b):
    c = lax.axis_index("c")
    s = lax.axis_index("s")
    wid = c * NS + s
    rpt = NPAD // NS

    pltpu.sync_copy(ga_hbm.at[pl.ds(s * rpt, rpt)], acc.at[pl.ds(s * rpt, rpt)])
    plsc.subcore_barrier()

    def gather(i, buf, sem):
        @pl.when(c == 0)
        def _():
            pltpu.async_copy(ga_hbm.at[sidx.at[i]], buf, sem)

        @pl.when(c == 1)
        def _():
            pltpu.async_copy(gb_hbm.at[sidx.at[i]], buf, sem)

    for h in range(CPW // CPW_H):
        pltpu.sync_copy(
            srcp_hbm.at[pl.ds(wid * CPW + h * CPW_H, CPW_H)], sidx
        )
        pltpu.sync_copy(
            dstp_hbm.at[pl.ds(wid * CPW + h * CPW_H, CPW_H)], didx
        )
        gather(0, rows_a, sem_a)

        def body(p, carry):
            i0 = 2 * p
            i1 = i0 + 1
            gather(i1, rows_b, sem_b)
            pltpu.make_async_copy(ga_hbm.at[sidx.at[i0]], rows_a, sem_a).wait()
            pltpu.sync_copy(rows_a, acc.at[didx.at[i0]], add=True)

            @pl.when(i1 + 1 < CPW_H)
            def _():
                gather(i1 + 1, rows_a, sem_a)

            pltpu.make_async_copy(ga_hbm.at[sidx.at[i1]], rows_b, sem_b).wait()
            pltpu.sync_copy(rows_b, acc.at[didx.at[i1]], add=True)
            return carry

        lax.fori_loop(0, CPW_H // 2, body, 0)
    plsc.subcore_barrier()
    pltpu.sync_copy(
        acc.at[pl.ds(s * rpt, rpt)], out_hbm.at[pl.ds(c * NPAD + s * rpt, rpt)]
    )


_agg_kernel = pl.kernel(
    _agg_body,
    out_type=jax.ShapeDtypeStruct((2 * NPAD, D), jnp.float32),
    mesh=_mesh,
    scratch_types=[
        pltpu.VMEM_SHARED((NPAD, D), jnp.float32),
        pltpu.VMEM((CPW_H, K), jnp.int32),
        pltpu.VMEM((CPW_H, K), jnp.int32),
        pltpu.VMEM((K, D), jnp.float32),
        pltpu.VMEM((K, D), jnp.float32),
        pltpu.SemaphoreType.DMA,
        pltpu.SemaphoreType.DMA,
    ],
    compiler_params=_sc_params,
)


def _dinv_of(degp):
    return lax.rsqrt(jnp.sum(degp, axis=1, keepdims=True) + 1.0)


def _gfirst_body(x_ref, w_ref, degp_ref, o_ref, o2_ref):
    dinv = _dinv_of(degp_ref[...])
    v = jnp.dot(
        x_ref[...], w_ref[...], preferred_element_type=jnp.float32
    ) * dinv
    o_ref[...] = v
    o2_ref[...] = v


def _combine_body(a0_ref, a1_ref, g_ref, degp_ref, b_ref, w_ref, o_ref, o2_ref):
    dinv = _dinv_of(degp_ref[...])
    pre = dinv * (a0_ref[...] + a1_ref[...] - g_ref[...]) + b_ref[...][None, :]
    xn = jnp.maximum(pre, 0.0)
    v = jnp.dot(
        xn, w_ref[...], preferred_element_type=jnp.float32
    ) * dinv
    o_ref[...] = v
    o2_ref[...] = v


def _final_body(a0_ref, a1_ref, g_ref, degp_ref, b_ref, wf1_ref, bf1_ref,
                wf2_ref, bf2_ref, o_ref):
    dinv = _dinv_of(degp_ref[...])
    pre = dinv * (a0_ref[...] + a1_ref[...] - g_ref[...]) + b_ref[...][None, :]
    x4 = jnp.maximum(pre, 0.0)
    f = jnp.dot(x4, wf1_ref[...], preferred_element_type=jnp.float32)
    f = jnp.maximum(f + bf1_ref[...][None, :], 0.0)
    o_ref[...] = jnp.dot(
        f, wf2_ref[...], preferred_element_type=jnp.float32
    ) + bf2_ref[...][None, :]


def _row_spec(off=0):
    return pl.BlockSpec((BN, D), lambda i, off=off: (i + off, 0))


def _full_spec(shape):
    nd = len(shape)
    return pl.BlockSpec(shape, lambda i: (0,) * nd)


_deg_spec = pl.BlockSpec((BN, NW), lambda i: (i, 0))
_nodes_shape = jax.ShapeDtypeStruct((NPAD, D), jnp.float32)

_gfirst = pl.pallas_call(
    _gfirst_body,
    grid=GRID,
    in_specs=[_row_spec(), _full_spec((D, D)), _deg_spec],
    out_specs=[_row_spec(), _row_spec()],
    out_shape=[_nodes_shape, _nodes_shape],
)

_combine = pl.pallas_call(
    _combine_body,
    grid=GRID,
    in_specs=[
        _row_spec(), _row_spec(NPAD // BN), _row_spec(), _deg_spec,
        _full_spec((D,)), _full_spec((D, D)),
    ],
    out_specs=[_row_spec(), _row_spec()],
    out_shape=[_nodes_shape, _nodes_shape],
)

_final = pl.pallas_call(
    _final_body,
    grid=GRID,
    in_specs=[
        _row_spec(), _row_spec(NPAD // BN), _row_spec(), _deg_spec,
        _full_spec((D,)), _full_spec((D, D)), _full_spec((D,)),
        _full_spec((D, D)), _full_spec((D,)),
    ],
    out_specs=_row_spec(),
    out_shape=_nodes_shape,
)


def kernel(x, edge_index, W1, b1, W2, b2, W3, b3, Wf1, bf1, Wf2, bf2):
    xp = jnp.pad(x, ((0, NPAD - N), (0, 0)))
    src = edge_index[0]
    dst = edge_index[1]
    npad_e = EP - E
    srcp = jnp.concatenate([src, jnp.zeros((npad_e,), src.dtype)])
    srcp = srcp.reshape(NW * CPW, K)
    dstp = jnp.concatenate(
        [dst, N + (jnp.arange(npad_e, dtype=dst.dtype) % (NPAD - N))]
    ).reshape(NW * CPW, K)

    degT = _deg_kernel(dst).reshape(NW, NPAD).T

    g1a, g1b = _gfirst(xp, W1, degT)
    acc = _agg_kernel(g1a, g1b, srcp, dstp)
    g2a, g2b = _combine(acc, acc, g1a, degT, b1, W2)
    acc = _agg_kernel(g2a, g2b, srcp, dstp)
    g3a, g3b = _combine(acc, acc, g2a, degT, b2, W3)
    acc = _agg_kernel(g3a, g3b, srcp, dstp)

    Wf2p = jnp.pad(Wf2, ((0, 0), (0, D - C)))
    bf2p = jnp.pad(bf2, (0, D - C))
    outp = _final(acc, acc, g3a, degT, b3, Wf1, bf1, Wf2p, bf2p)
    return outp[:N, :C]


# speedup vs baseline: 1.2525x; 1.2525x over previous
import jax
import jax.numpy as jnp
from jax import lax
from jax.experimental import pallas as pl
from jax.experimental.pallas import tpu as pltpu
from jax.experimental.pallas import tpu_sc as plsc

N = 10000
E = 320000
D = 128
C = 40

NC = 2
NS = 16
NW = NC * NS

NPAD = 10240
K = 128
CPW0 = 128
CPW1 = 32
SLAB = 32
NCH = NS * (CPW0 + CPW1)
EP = NCH * K

DEG_CHUNK = 2000
EPW_DEG = E // NW

BN = 1024
GRID = (NPAD // BN,)

_mesh = plsc.VectorSubcoreMesh(
    core_axis_name="c", subcore_axis_name="s", num_cores=NC, num_subcores=NS
)
_sc_params = pltpu.CompilerParams(needs_layout_passes=False)


def _deg_body(dst_hbm, out_hbm, dacc, dchunk):
    c = lax.axis_index("c")
    s = lax.axis_index("s")
    wid = c * NS + s
    zeros16 = jnp.zeros((16,), jnp.float32)
    ones16 = jnp.ones((16,), jnp.float32)

    def zb(i, carry):
        dacc[pl.ds(i * 16, 16)] = zeros16
        return carry

    lax.fori_loop(0, NPAD // 16, zb, 0)

    def cb(ci, carry):
        base = wid * EPW_DEG + ci * DEG_CHUNK
        pltpu.sync_copy(dst_hbm.at[pl.ds(base, DEG_CHUNK)], dchunk)

        def ib(j, carry2):
            idx = dchunk[pl.ds(j * 16, 16)]
            plsc.addupdate_scatter(dacc, [idx], ones16)
            return carry2

        lax.fori_loop(0, DEG_CHUNK // 16, ib, 0)
        return carry

    lax.fori_loop(0, EPW_DEG // DEG_CHUNK, cb, 0)
    pltpu.sync_copy(dacc, out_hbm.at[pl.ds(wid * NPAD, NPAD)])


_deg_kernel = pl.kernel(
    _deg_body,
    out_type=jax.ShapeDtypeStruct((NW * NPAD,), jnp.float32),
    mesh=_mesh,
    scratch_types=[
        pltpu.VMEM((NPAD,), jnp.float32),
        pltpu.VMEM((DEG_CHUNK,), jnp.int32),
    ],
    compiler_params=_sc_params,
)


def _agg_body(g_hbm, srcp_hbm, dstp_hbm, out_hbm, acc, sidx, didx,
              rows_a, rows_b, sem_a, sem_b):
    c = lax.axis_index("c")
    s = lax.axis_index("s")
    wid = c * NS + s
    rpt = NPAD // NS

    pltpu.sync_copy(g_hbm.at[pl.ds(s * rpt, rpt)], acc.at[pl.ds(s * rpt, rpt)])
    plsc.subcore_barrier()

    def gather(i, buf, sem):
        return pltpu.async_copy(g_hbm.at[sidx.at[i]], buf, sem)

    def run_slab(chunk0):
        pltpu.sync_copy(srcp_hbm.at[pl.ds(chunk0, SLAB)], sidx)
        pltpu.sync_copy(dstp_hbm.at[pl.ds(chunk0, SLAB)], didx)
        gather(0, rows_a, sem_a)

        def body(p, carry):
            i0 = 2 * p
            i1 = i0 + 1
            gather(i1, rows_b, sem_b)
            pltpu.make_async_copy(g_hbm.at[sidx.at[i0]], rows_a, sem_a).wait()
            pltpu.sync_copy(rows_a, acc.at[didx.at[i0]], add=True)

            @pl.when(i1 + 1 < SLAB)
            def _():
                gather(i1 + 1, rows_a, sem_a)

            pltpu.make_async_copy(g_hbm.at[sidx.at[i1]], rows_b, sem_b).wait()
            pltpu.sync_copy(rows_b, acc.at[didx.at[i1]], add=True)
            return carry

        lax.fori_loop(0, SLAB // 2, body, 0)

    @pl.when(c == 0)
    def _():
        for r in range(CPW0 // SLAB):
            run_slab(s * CPW0 + r * SLAB)

    @pl.when(c == 1)
    def _():
        for r in range(CPW1 // SLAB):
            run_slab(NS * CPW0 + s * CPW1 + r * SLAB)

    plsc.subcore_barrier()
    pltpu.sync_copy(
        acc.at[pl.ds(s * rpt, rpt)], out_hbm.at[pl.ds(c * NPAD + s * rpt, rpt)]
    )


_agg_kernel = pl.kernel(
    _agg_body,
    out_type=jax.ShapeDtypeStruct((2 * NPAD, D), jnp.float32),
    mesh=_mesh,
    scratch_types=[
        pltpu.VMEM_SHARED((NPAD, D), jnp.float32),
        pltpu.VMEM((SLAB, K), jnp.int32),
        pltpu.VMEM((SLAB, K), jnp.int32),
        pltpu.VMEM((K, D), jnp.float32),
        pltpu.VMEM((K, D), jnp.float32),
        pltpu.SemaphoreType.DMA,
        pltpu.SemaphoreType.DMA,
    ],
    compiler_params=_sc_params,
)


def _dinv_of(degp):
    return lax.rsqrt(jnp.sum(degp, axis=1, keepdims=True) + 1.0)


def _gfirst_body(x_ref, w_ref, degp_ref, o_ref):
    dinv = _dinv_of(degp_ref[...])
    o_ref[...] = jnp.dot(
        x_ref[...], w_ref[...], preferred_element_type=jnp.float32
    ) * dinv


def _combine_body(a0_ref, a1_ref, g_ref, degp_ref, b_ref, w_ref, o_ref):
    dinv = _dinv_of(degp_ref[...])
    pre = dinv * (a0_ref[...] + a1_ref[...] - g_ref[...]) + b_ref[...][None, :]
    xn = jnp.maximum(pre, 0.0)
    o_ref[...] = jnp.dot(
        xn, w_ref[...], preferred_element_type=jnp.float32
    ) * dinv


def _final_body(a0_ref, a1_ref, g_ref, degp_ref, b_ref, wf1_ref, bf1_ref,
                wf2_ref, bf2_ref, o_ref):
    dinv = _dinv_of(degp_ref[...])
    pre = dinv * (a0_ref[...] + a1_ref[...] - g_ref[...]) + b_ref[...][None, :]
    x4 = jnp.maximum(pre, 0.0)
    f = jnp.dot(x4, wf1_ref[...], preferred_element_type=jnp.float32)
    f = jnp.maximum(f + bf1_ref[...][None, :], 0.0)
    o_ref[...] = jnp.dot(
        f, wf2_ref[...], preferred_element_type=jnp.float32
    ) + bf2_ref[...][None, :]


def _row_spec(off=0):
    return pl.BlockSpec((BN, D), lambda i, off=off: (i + off, 0))


def _full_spec(shape):
    nd = len(shape)
    return pl.BlockSpec(shape, lambda i: (0,) * nd)


_deg_spec = pl.BlockSpec((BN, NW), lambda i: (i, 0))
_nodes_shape = jax.ShapeDtypeStruct((NPAD, D), jnp.float32)

_gfirst = pl.pallas_call(
    _gfirst_body,
    grid=GRID,
    in_specs=[_row_spec(), _full_spec((D, D)), _deg_spec],
    out_specs=_row_spec(),
    out_shape=_nodes_shape,
)

_combine = pl.pallas_call(
    _combine_body,
    grid=GRID,
    in_specs=[
        _row_spec(), _row_spec(NPAD // BN), _row_spec(), _deg_spec,
        _full_spec((D,)), _full_spec((D, D)),
    ],
    out_specs=_row_spec(),
    out_shape=_nodes_shape,
)

_final = pl.pallas_call(
    _final_body,
    grid=GRID,
    in_specs=[
        _row_spec(), _row_spec(NPAD // BN), _row_spec(), _deg_spec,
        _full_spec((D,)), _full_spec((D, D)), _full_spec((D,)),
        _full_spec((D, D)), _full_spec((D,)),
    ],
    out_specs=_row_spec(),
    out_shape=_nodes_shape,
)


def kernel(x, edge_index, W1, b1, W2, b2, W3, b3, Wf1, bf1, Wf2, bf2):
    xp = jnp.pad(x, ((0, NPAD - N), (0, 0)))
    src = edge_index[0]
    dst = edge_index[1]
    npad_e = EP - E
    srcp = jnp.concatenate([src, jnp.zeros((npad_e,), src.dtype)])
    srcp = srcp.reshape(NCH, K)
    dstp = jnp.concatenate(
        [dst, N + (jnp.arange(npad_e, dtype=dst.dtype) % (NPAD - N))]
    ).reshape(NCH, K)

    degT = _deg_kernel(dst).reshape(NW, NPAD).T

    g1 = _gfirst(xp, W1, degT)
    acc = _agg_kernel(g1, srcp, dstp)
    g2 = _combine(acc, acc, g1, degT, b1, W2)
    acc = _agg_kernel(g2, srcp, dstp)
    g3 = _combine(acc, acc, g2, degT, b2, W3)
    acc = _agg_kernel(g3, srcp, dstp)

    Wf2p = jnp.pad(Wf2, ((0, 0), (0, D - C)))
    bf2p = jnp.pad(bf2, (0, D - C))
    outp = _final(acc, acc, g3, degT, b3, Wf1, bf1, Wf2p, bf2p)
    return outp[:N, :C]


# speedup vs baseline: 1.3891x; 1.1091x over previous
import jax
import jax.numpy as jnp
from jax import lax
from jax.experimental import pallas as pl
from jax.experimental.pallas import tpu as pltpu
from jax.experimental.pallas import tpu_sc as plsc

N = 10000
E = 320000
D = 128
C = 40

NC = 2
NS = 16
NW = NC * NS

NPAD = 10240
K = 128
CPW0 = 144
CPW1 = 16
SLAB = 16
NCH = NS * (CPW0 + CPW1)
EP = NCH * K

DEG_CHUNK = 2000
EPW_DEG = E // NW

BN = 1024
GRID = (NPAD // BN,)

_mesh = plsc.VectorSubcoreMesh(
    core_axis_name="c", subcore_axis_name="s", num_cores=NC, num_subcores=NS
)
_sc_params = pltpu.CompilerParams(needs_layout_passes=False)


def _deg_body(dst_hbm, out_hbm, dacc, dchunk):
    c = lax.axis_index("c")
    s = lax.axis_index("s")
    wid = c * NS + s
    zeros16 = jnp.zeros((16,), jnp.float32)
    ones16 = jnp.ones((16,), jnp.float32)

    def zb(i, carry):
        dacc[pl.ds(i * 16, 16)] = zeros16
        return carry

    lax.fori_loop(0, NPAD // 16, zb, 0)

    def cb(ci, carry):
        base = wid * EPW_DEG + ci * DEG_CHUNK
        pltpu.sync_copy(dst_hbm.at[pl.ds(base, DEG_CHUNK)], dchunk)

        def ib(j, carry2):
            idx = dchunk[pl.ds(j * 16, 16)]
            plsc.addupdate_scatter(dacc, [idx], ones16)
            return carry2

        lax.fori_loop(0, DEG_CHUNK // 16, ib, 0)
        return carry

    lax.fori_loop(0, EPW_DEG // DEG_CHUNK, cb, 0)
    pltpu.sync_copy(dacc, out_hbm.at[pl.ds(wid * NPAD, NPAD)])


_deg_kernel = pl.kernel(
    _deg_body,
    out_type=jax.ShapeDtypeStruct((NW * NPAD,), jnp.float32),
    mesh=_mesh,
    scratch_types=[
        pltpu.VMEM((NPAD,), jnp.float32),
        pltpu.VMEM((DEG_CHUNK,), jnp.int32),
    ],
    compiler_params=_sc_params,
)


def _agg_body(g_hbm, srcp_hbm, dstp_hbm, out_hbm, acc, sidx, didx,
              rows_a, rows_b, sem_a, sem_b):
    c = lax.axis_index("c")
    s = lax.axis_index("s")
    wid = c * NS + s
    rpt = NPAD // NS

    z16 = jnp.zeros((16,), jnp.float32)

    def zb(r, carry):
        for j in range(D // 16):
            rows_a[r, pl.ds(j * 16, 16)] = z16
        return carry

    lax.fori_loop(0, K, zb, 0)
    for t in range(rpt // K):
        pltpu.sync_copy(rows_a, acc.at[pl.ds(s * rpt + t * K, K)])
    plsc.subcore_barrier()

    def gather(i, buf, sem):
        return pltpu.async_copy(g_hbm.at[sidx.at[i]], buf, sem)

    def run_slab(chunk0):
        pltpu.sync_copy(srcp_hbm.at[pl.ds(chunk0, SLAB)], sidx)
        pltpu.sync_copy(dstp_hbm.at[pl.ds(chunk0, SLAB)], didx)
        gather(0, rows_a, sem_a)

        def body(p, carry):
            i0 = 2 * p
            i1 = i0 + 1
            gather(i1, rows_b, sem_b)
            pltpu.make_async_copy(g_hbm.at[sidx.at[i0]], rows_a, sem_a).wait()
            pltpu.sync_copy(rows_a, acc.at[didx.at[i0]], add=True)

            @pl.when(i1 + 1 < SLAB)
            def _():
                gather(i1 + 1, rows_a, sem_a)

            pltpu.make_async_copy(g_hbm.at[sidx.at[i1]], rows_b, sem_b).wait()
            pltpu.sync_copy(rows_b, acc.at[didx.at[i1]], add=True)
            return carry

        lax.fori_loop(0, SLAB // 2, body, 0)

    @pl.when(c == 0)
    def _():
        for r in range(CPW0 // SLAB):
            run_slab(s * CPW0 + r * SLAB)

    @pl.when(c == 1)
    def _():
        for r in range(CPW1 // SLAB):
            run_slab(NS * CPW0 + s * CPW1 + r * SLAB)

    plsc.subcore_barrier()
    pltpu.sync_copy(
        acc.at[pl.ds(s * rpt, rpt)], out_hbm.at[pl.ds(c * NPAD + s * rpt, rpt)]
    )


_agg_kernel = pl.kernel(
    _agg_body,
    out_type=jax.ShapeDtypeStruct((2 * NPAD, D), jnp.float32),
    mesh=_mesh,
    scratch_types=[
        pltpu.VMEM_SHARED((NPAD, D), jnp.float32),
        pltpu.VMEM((SLAB, K), jnp.int32),
        pltpu.VMEM((SLAB, K), jnp.int32),
        pltpu.VMEM((K, D), jnp.float32),
        pltpu.VMEM((K, D), jnp.float32),
        pltpu.SemaphoreType.DMA,
        pltpu.SemaphoreType.DMA,
    ],
    compiler_params=_sc_params,
)


def _dinv_of(degp):
    return lax.rsqrt(jnp.sum(degp, axis=1, keepdims=True) + 1.0)


def _gfirst_body(x_ref, w_ref, degp_ref, o_ref):
    dinv = _dinv_of(degp_ref[...])
    o_ref[...] = jnp.dot(
        x_ref[...], w_ref[...], preferred_element_type=jnp.float32
    ) * dinv


def _combine_body(a0_ref, a1_ref, g_ref, degp_ref, b_ref, w_ref, o_ref):
    dinv = _dinv_of(degp_ref[...])
    pre = dinv * (a0_ref[...] + a1_ref[...] + g_ref[...]) + b_ref[...][None, :]
    xn = jnp.maximum(pre, 0.0)
    o_ref[...] = jnp.dot(
        xn, w_ref[...], preferred_element_type=jnp.float32
    ) * dinv


def _final_body(a0_ref, a1_ref, g_ref, degp_ref, b_ref, wf1_ref, bf1_ref,
                wf2_ref, bf2_ref, o_ref):
    dinv = _dinv_of(degp_ref[...])
    pre = dinv * (a0_ref[...] + a1_ref[...] + g_ref[...]) + b_ref[...][None, :]
    x4 = jnp.maximum(pre, 0.0)
    f = jnp.dot(x4, wf1_ref[...], preferred_element_type=jnp.float32)
    f = jnp.maximum(f + bf1_ref[...][None, :], 0.0)
    o_ref[...] = jnp.dot(
        f, wf2_ref[...], preferred_element_type=jnp.float32
    ) + bf2_ref[...][None, :]


def _row_spec(off=0):
    return pl.BlockSpec((BN, D), lambda i, off=off: (i + off, 0))


def _full_spec(shape):
    nd = len(shape)
    return pl.BlockSpec(shape, lambda i: (0,) * nd)


_deg_spec = pl.BlockSpec((BN, NW), lambda i: (i, 0))
_nodes_shape = jax.ShapeDtypeStruct((NPAD, D), jnp.float32)

_gfirst = pl.pallas_call(
    _gfirst_body,
    grid=GRID,
    in_specs=[_row_spec(), _full_spec((D, D)), _deg_spec],
    out_specs=_row_spec(),
    out_shape=_nodes_shape,
)

_combine = pl.pallas_call(
    _combine_body,
    grid=GRID,
    in_specs=[
        _row_spec(), _row_spec(NPAD // BN), _row_spec(), _deg_spec,
        _full_spec((D,)), _full_spec((D, D)),
    ],
    out_specs=_row_spec(),
    out_shape=_nodes_shape,
)

_final = pl.pallas_call(
    _final_body,
    grid=GRID,
    in_specs=[
        _row_spec(), _row_spec(NPAD // BN), _row_spec(), _deg_spec,
        _full_spec((D,)), _full_spec((D, D)), _full_spec((D,)),
        _full_spec((D, D)), _full_spec((D,)),
    ],
    out_specs=_row_spec(),
    out_shape=_nodes_shape,
)


def kernel(x, edge_index, W1, b1, W2, b2, W3, b3, Wf1, bf1, Wf2, bf2):
    xp = jnp.pad(x, ((0, NPAD - N), (0, 0)))
    src = edge_index[0]
    dst = edge_index[1]
    npad_e = EP - E
    srcp = jnp.concatenate([src, jnp.zeros((npad_e,), src.dtype)])
    srcp = srcp.reshape(NCH, K)
    dstp = jnp.concatenate(
        [dst, N + (jnp.arange(npad_e, dtype=dst.dtype) % (NPAD - N))]
    ).reshape(NCH, K)

    degT = _deg_kernel(dst).reshape(NW, NPAD).T

    g1 = _gfirst(xp, W1, degT)
    acc = _agg_kernel(g1, srcp, dstp)
    g2 = _combine(acc, acc, g1, degT, b1, W2)
    acc = _agg_kernel(g2, srcp, dstp)
    g3 = _combine(acc, acc, g2, degT, b2, W3)
    acc = _agg_kernel(g3, srcp, dstp)

    Wf2p = jnp.pad(Wf2, ((0, 0), (0, D - C)))
    bf2p = jnp.pad(bf2, (0, D - C))
    outp = _final(acc, acc, g3, degT, b3, Wf1, bf1, Wf2p, bf2p)
    return outp[:N, :C]


# speedup vs baseline: 1.3970x; 1.0057x over previous
import jax
import jax.numpy as jnp
from jax import lax
from jax.experimental import pallas as pl
from jax.experimental.pallas import tpu as pltpu
from jax.experimental.pallas import tpu_sc as plsc

N = 10000
E = 320000
D = 128
C = 40

NC = 2
NS = 16
NW = NC * NS

NPAD = 10240
K = 128
CPW0 = 152
CPW1 = 8
SLAB = 8
NCH = NS * (CPW0 + CPW1)
EP = NCH * K

DEG_CHUNK = 2000
EPW_DEG = E // NW

BN = 1024
GRID = (NPAD // BN,)

_mesh = plsc.VectorSubcoreMesh(
    core_axis_name="c", subcore_axis_name="s", num_cores=NC, num_subcores=NS
)
_sc_params = pltpu.CompilerParams(needs_layout_passes=False)


def _deg_body(dst_hbm, out_hbm, dacc, dchunk):
    c = lax.axis_index("c")
    s = lax.axis_index("s")
    wid = c * NS + s
    zeros16 = jnp.zeros((16,), jnp.float32)
    ones16 = jnp.ones((16,), jnp.float32)

    def zb(i, carry):
        dacc[pl.ds(i * 16, 16)] = zeros16
        return carry

    lax.fori_loop(0, NPAD // 16, zb, 0)

    def cb(ci, carry):
        base = wid * EPW_DEG + ci * DEG_CHUNK
        pltpu.sync_copy(dst_hbm.at[pl.ds(base, DEG_CHUNK)], dchunk)

        def ib(j, carry2):
            idx = dchunk[pl.ds(j * 16, 16)]
            plsc.addupdate_scatter(dacc, [idx], ones16)
            return carry2

        lax.fori_loop(0, DEG_CHUNK // 16, ib, 0)
        return carry

    lax.fori_loop(0, EPW_DEG // DEG_CHUNK, cb, 0)
    pltpu.sync_copy(dacc, out_hbm.at[pl.ds(wid * NPAD, NPAD)])


_deg_kernel = pl.kernel(
    _deg_body,
    out_type=jax.ShapeDtypeStruct((NW * NPAD,), jnp.float32),
    mesh=_mesh,
    scratch_types=[
        pltpu.VMEM((NPAD,), jnp.float32),
        pltpu.VMEM((DEG_CHUNK,), jnp.int32),
    ],
    compiler_params=_sc_params,
)


def _agg_body(g_hbm, srcp_hbm, dstp_hbm, out_hbm, acc, sidx, didx,
              rows_a, rows_b, sem_a, sem_b):
    c = lax.axis_index("c")
    s = lax.axis_index("s")
    wid = c * NS + s
    rpt = NPAD // NS

    z16 = jnp.zeros((16,), jnp.float32)

    def zb(r, carry):
        for j in range(D // 16):
            rows_a[r, pl.ds(j * 16, 16)] = z16
        return carry

    lax.fori_loop(0, K, zb, 0)
    for t in range(rpt // K):
        pltpu.sync_copy(rows_a, acc.at[pl.ds(s * rpt + t * K, K)])
    plsc.subcore_barrier()

    def gather(i, buf, sem):
        return pltpu.async_copy(g_hbm.at[sidx.at[i]], buf, sem)

    def run_slab(chunk0):
        pltpu.sync_copy(srcp_hbm.at[pl.ds(chunk0, SLAB)], sidx)
        pltpu.sync_copy(dstp_hbm.at[pl.ds(chunk0, SLAB)], didx)
        gather(0, rows_a, sem_a)

        def body(p, carry):
            i0 = 2 * p
            i1 = i0 + 1
            gather(i1, rows_b, sem_b)
            pltpu.make_async_copy(g_hbm.at[sidx.at[i0]], rows_a, sem_a).wait()
            pltpu.sync_copy(rows_a, acc.at[didx.at[i0]], add=True)

            @pl.when(i1 + 1 < SLAB)
            def _():
                gather(i1 + 1, rows_a, sem_a)

            pltpu.make_async_copy(g_hbm.at[sidx.at[i1]], rows_b, sem_b).wait()
            pltpu.sync_copy(rows_b, acc.at[didx.at[i1]], add=True)
            return carry

        lax.fori_loop(0, SLAB // 2, body, 0)

    @pl.when(c == 0)
    def _():
        for r in range(CPW0 // SLAB):
            run_slab(s * CPW0 + r * SLAB)

    @pl.when(c == 1)
    def _():
        for r in range(CPW1 // SLAB):
            run_slab(NS * CPW0 + s * CPW1 + r * SLAB)

    plsc.subcore_barrier()
    pltpu.sync_copy(
        acc.at[pl.ds(s * rpt, rpt)], out_hbm.at[pl.ds(c * NPAD + s * rpt, rpt)]
    )


_agg_kernel = pl.kernel(
    _agg_body,
    out_type=jax.ShapeDtypeStruct((2 * NPAD, D), jnp.float32),
    mesh=_mesh,
    scratch_types=[
        pltpu.VMEM_SHARED((NPAD, D), jnp.float32),
        pltpu.VMEM((SLAB, K), jnp.int32),
        pltpu.VMEM((SLAB, K), jnp.int32),
        pltpu.VMEM((K, D), jnp.float32),
        pltpu.VMEM((K, D), jnp.float32),
        pltpu.SemaphoreType.DMA,
        pltpu.SemaphoreType.DMA,
    ],
    compiler_params=_sc_params,
)


def _dinv_of(degp):
    return lax.rsqrt(jnp.sum(degp, axis=1, keepdims=True) + 1.0)


def _gfirst_body(x_ref, w_ref, degp_ref, o_ref):
    dinv = _dinv_of(degp_ref[...])
    o_ref[...] = jnp.dot(
        x_ref[...], w_ref[...], preferred_element_type=jnp.float32
    ) * dinv


def _combine_body(a0_ref, a1_ref, g_ref, degp_ref, b_ref, w_ref, o_ref):
    dinv = _dinv_of(degp_ref[...])
    pre = dinv * (a0_ref[...] + a1_ref[...] + g_ref[...]) + b_ref[...][None, :]
    xn = jnp.maximum(pre, 0.0)
    o_ref[...] = jnp.dot(
        xn, w_ref[...], preferred_element_type=jnp.float32
    ) * dinv


def _final_body(a0_ref, a1_ref, g_ref, degp_ref, b_ref, wf1_ref, bf1_ref,
                wf2_ref, bf2_ref, o_ref):
    dinv = _dinv_of(degp_ref[...])
    pre = dinv * (a0_ref[...] + a1_ref[...] + g_ref[...]) + b_ref[...][None, :]
    x4 = jnp.maximum(pre, 0.0)
    f = jnp.dot(x4, wf1_ref[...], preferred_element_type=jnp.float32)
    f = jnp.maximum(f + bf1_ref[...][None, :], 0.0)
    o_ref[...] = jnp.dot(
        f, wf2_ref[...], preferred_element_type=jnp.float32
    ) + bf2_ref[...][None, :]


def _row_spec(off=0):
    return pl.BlockSpec((BN, D), lambda i, off=off: (i + off, 0))


def _full_spec(shape):
    nd = len(shape)
    return pl.BlockSpec(shape, lambda i: (0,) * nd)


_deg_spec = pl.BlockSpec((BN, NW), lambda i: (i, 0))
_nodes_shape = jax.ShapeDtypeStruct((NPAD, D), jnp.float32)

_gfirst = pl.pallas_call(
    _gfirst_body,
    grid=GRID,
    in_specs=[_row_spec(), _full_spec((D, D)), _deg_spec],
    out_specs=_row_spec(),
    out_shape=_nodes_shape,
)

_combine = pl.pallas_call(
    _combine_body,
    grid=GRID,
    in_specs=[
        _row_spec(), _row_spec(NPAD // BN), _row_spec(), _deg_spec,
        _full_spec((D,)), _full_spec((D, D)),
    ],
    out_specs=_row_spec(),
    out_shape=_nodes_shape,
)

_final = pl.pallas_call(
    _final_body,
    grid=GRID,
    in_specs=[
        _row_spec(), _row_spec(NPAD // BN), _row_spec(), _deg_spec,
        _full_spec((D,)), _full_spec((D, D)), _full_spec((D,)),
        _full_spec((D, D)), _full_spec((D,)),
    ],
    out_specs=_row_spec(),
    out_shape=_nodes_shape,
)


def kernel(x, edge_index, W1, b1, W2, b2, W3, b3, Wf1, bf1, Wf2, bf2):
    xp = jnp.pad(x, ((0, NPAD - N), (0, 0)))
    src = edge_index[0]
    dst = edge_index[1]
    npad_e = EP - E
    srcp = jnp.concatenate([src, jnp.zeros((npad_e,), src.dtype)])
    srcp = srcp.reshape(NCH, K)
    dstp = jnp.concatenate(
        [dst, N + (jnp.arange(npad_e, dtype=dst.dtype) % (NPAD - N))]
    ).reshape(NCH, K)

    degT = _deg_kernel(dst).reshape(NW, NPAD).T

    g1 = _gfirst(xp, W1, degT)
    acc = _agg_kernel(g1, srcp, dstp)
    g2 = _combine(acc, acc, g1, degT, b1, W2)
    acc = _agg_kernel(g2, srcp, dstp)
    g3 = _combine(acc, acc, g2, degT, b2, W3)
    acc = _agg_kernel(g3, srcp, dstp)

    Wf2p = jnp.pad(Wf2, ((0, 0), (0, D - C)))
    bf2p = jnp.pad(bf2, (0, D - C))
    outp = _final(acc, acc, g3, degT, b3, Wf1, bf1, Wf2p, bf2p)
    return outp[:N, :C]
